# baseline (speedup 1.0000x reference)
import functools

import jax
import jax.numpy as jnp
from jax import lax
from jax.experimental import pallas as pl
from jax.experimental.pallas import tpu as pltpu
from jax.experimental.pallas import tpu_sc as plsc

_B = 16384
_L = 16


def _make_sc_kernel():
    info = plsc.get_sparse_core_info()
    nc, ns = info.num_cores, info.num_subcores
    nw = nc * ns
    chunk = _B // nw
    nv = chunk // _L

    mesh = plsc.VectorSubcoreMesh(core_axis_name="c", subcore_axis_name="s")

    @functools.partial(
        pl.kernel,
        mesh=mesh,
        out_type=jax.ShapeDtypeStruct((_B,), jnp.float32),
        scratch_types=[
            pltpu.VMEM((chunk,), jnp.float32),
            pltpu.VMEM((chunk,), jnp.float32),
            pltpu.VMEM((chunk,), jnp.float32),
            pltpu.VMEM((3 * _L,), jnp.float32),
            pltpu.SemaphoreType.DMA,
            pltpu.SemaphoreType.DMA,
            pltpu.SemaphoreType.DMA,
        ],
    )
    def sc_kernel(g_hbm, label_hbm, params_hbm, out_hbm,
                  g_v, l_v, o_v, p_v, sem_g, sem_l, sem_p):
        wid = lax.axis_index("s") * nc + lax.axis_index("c")
        base = wid * chunk
        cp_g = pltpu.async_copy(g_hbm.at[pl.ds(base, chunk)], g_v, sem_g)
        cp_l = pltpu.async_copy(label_hbm.at[pl.ds(base, chunk)], l_v, sem_l)
        cp_p = pltpu.async_copy(params_hbm, p_v, sem_p)
        cp_p.wait()
        a = p_v[pl.ds(0, _L)]
        b = p_v[pl.ds(_L, _L)]
        c = p_v[pl.ds(2 * _L, _L)]
        cp_g.wait()
        cp_l.wait()
        for i in range(nv):
            g = g_v[pl.ds(i * _L, _L)]
            lab = l_v[pl.ds(i * _L, _L)]
            o_v[pl.ds(i * _L, _L)] = a + b * g + c * (lab * g)
        pltpu.sync_copy(o_v, out_hbm.at[pl.ds(base, chunk)])

    return sc_kernel


_sc_kernel = _make_sc_kernel()


def kernel(user, item, g_s, label, alpha, beta, label_coef):
    params = jnp.concatenate([
        jnp.broadcast_to(alpha, (_L,)),
        jnp.broadcast_to(beta, (_L,)),
        jnp.broadcast_to(label_coef, (_L,)),
    ])
    return _sc_kernel(g_s, label, params)


# speedup vs baseline: 2.4340x; 2.4340x over previous
import jax
import jax.numpy as jnp
from jax.experimental import pallas as pl
from jax.experimental.pallas import tpu as pltpu

_B = 16384
_GRID = 4
_COLS = 512
_ROWS = _B // _COLS
_BLK = _ROWS // _GRID


def _body(alpha_ref, beta_ref, lc_ref, g_ref, label_ref, out_ref):
    a = alpha_ref[0]
    b = beta_ref[0]
    c = lc_ref[0]
    g = g_ref[...]
    out_ref[...] = a + b * g + c * (label_ref[...] * g)


def kernel(user, item, g_s, label, alpha, beta, label_coef):
    g2 = g_s.reshape(_ROWS, _COLS)
    l2 = label.reshape(_ROWS, _COLS)
    blk = pl.BlockSpec((_BLK, _COLS), lambda i: (i, 0))
    out = pl.pallas_call(
        _body,
        grid=(_GRID,),
        out_shape=jax.ShapeDtypeStruct(g2.shape, jnp.float32),
        in_specs=[
            pl.BlockSpec(memory_space=pltpu.SMEM),
            pl.BlockSpec(memory_space=pltpu.SMEM),
            pl.BlockSpec(memory_space=pltpu.SMEM),
            blk,
            blk,
        ],
        out_specs=blk,
        compiler_params=pltpu.CompilerParams(
            dimension_semantics=("arbitrary",),
        ),
    )(alpha, beta, label_coef, g2, l2)
    return out.reshape(_B)


# speedup vs baseline: 5.9810x; 2.4573x over previous
import jax
import jax.numpy as jnp
from jax.experimental import pallas as pl
from jax.experimental.pallas import tpu as pltpu


def _body(alpha_ref, beta_ref, lc_ref, g_ref, label_ref, out_ref):
    a = alpha_ref[0]
    b = beta_ref[0]
    c = lc_ref[0]
    g = g_ref[...]
    out_ref[...] = a + b * g + c * (label_ref[...] * g)


def kernel(user, item, g_s, label, alpha, beta, label_coef):
    return pl.pallas_call(
        _body,
        out_shape=jax.ShapeDtypeStruct(g_s.shape, jnp.float32),
        in_specs=[
            pl.BlockSpec(memory_space=pltpu.SMEM),
            pl.BlockSpec(memory_space=pltpu.SMEM),
            pl.BlockSpec(memory_space=pltpu.SMEM),
            pl.BlockSpec(memory_space=pltpu.VMEM),
            pl.BlockSpec(memory_space=pltpu.VMEM),
        ],
        out_specs=pl.BlockSpec(memory_space=pltpu.VMEM),
    )(alpha, beta, label_coef, g_s, label)


# speedup vs baseline: 5.9936x; 1.0021x over previous
import jax
import jax.numpy as jnp
from jax.experimental import pallas as pl
from jax.experimental.pallas import tpu as pltpu


def _body(alpha_ref, beta_ref, lc_ref, g_ref, label_ref, out_ref):
    a = alpha_ref[0]
    b = beta_ref[0]
    c = lc_ref[0]
    g = g_ref[...]
    out_ref[...] = a + b * g + c * (label_ref[...] * g)


def kernel(user, item, g_s, label, alpha, beta, label_coef):
    half = g_s.shape[0] // 2
    blk = pl.BlockSpec((half,), lambda i: (i,))
    return pl.pallas_call(
        _body,
        grid=(2,),
        out_shape=jax.ShapeDtypeStruct(g_s.shape, jnp.float32),
        in_specs=[
            pl.BlockSpec(memory_space=pltpu.SMEM),
            pl.BlockSpec(memory_space=pltpu.SMEM),
            pl.BlockSpec(memory_space=pltpu.SMEM),
            blk,
            blk,
        ],
        out_specs=blk,
    )(alpha, beta, label_coef, g_s, label)
